# baseline (speedup 1.0000x reference)
import jax
import jax.numpy as jnp
from jax import lax
from jax.experimental import pallas as pl
from jax.experimental.pallas import tpu as pltpu

_BM = 512


def _gemm_bias_kernel(x_ref, w_ref, b_ref, o_ref):
    x = x_ref[...]
    acc = lax.transpose(b_ref[...], (1, 0))
    for k in range(w_ref.shape[1]):
        acc = acc + lax.dot_general(
            w_ref[:, k, :],
            x[:, 128 * k : 128 * (k + 1)],
            dimension_numbers=(((1,), (1,)), ((), ())),
            preferred_element_type=jnp.float32,
        )
    o_ref[...] = acc


def kernel(x, W, b):
    B, D = x.shape
    E = W.shape[0]
    w2 = W.reshape(E, D // 128, 128)
    b_row = b.reshape(1, E)
    out_t = pl.pallas_call(
        _gemm_bias_kernel,
        grid=(B // _BM,),
        in_specs=[
            pl.BlockSpec((_BM, D), lambda i: (i, 0)),
            pl.BlockSpec((E, D // 128, 128), lambda i: (0, 0, 0)),
            pl.BlockSpec((1, E), lambda i: (0, 0)),
        ],
        out_specs=pl.BlockSpec((E, _BM), lambda i: (0, i)),
        out_shape=jax.ShapeDtypeStruct((E, B), jnp.float32),
        compiler_params=pltpu.CompilerParams(
            dimension_semantics=("arbitrary",),
        ),
    )(x, w2, b_row)
    return out_t.T


# speedup vs baseline: 1.0907x; 1.0907x over previous
import jax
import jax.numpy as jnp
from jax import lax
from jax.experimental import pallas as pl
from jax.experimental.pallas import tpu as pltpu

_BM = 1024


def _gemm_chunks(x, w_ref, bias_t):
    acc = bias_t
    for k in range(w_ref.shape[1]):
        acc = acc + lax.dot_general(
            w_ref[:, k, :],
            x[:, 128 * k : 128 * (k + 1)],
            dimension_numbers=(((1,), (1,)), ((), ())),
            preferred_element_type=jnp.float32,
        )
    return acc


def _gemm_bias_kernel(xa_ref, xb_ref, w_ref, b_ref, o_ref):
    bias_t = lax.transpose(b_ref[...], (1, 0))
    o_ref[:, :_BM] = _gemm_chunks(xa_ref[...], w_ref, bias_t)
    o_ref[:, _BM:] = _gemm_chunks(xb_ref[...], w_ref, bias_t)


def kernel(x, W, b):
    B, D = x.shape
    E = W.shape[0]
    x3 = x.reshape(B // _BM, _BM, D)
    w2 = W.reshape(E, D // 128, 128)
    b_row = b.reshape(1, E)
    out_t = pl.pallas_call(
        _gemm_bias_kernel,
        grid=(B // (2 * _BM),),
        in_specs=[
            pl.BlockSpec((None, _BM, D), lambda i: (2 * i, 0, 0)),
            pl.BlockSpec((None, _BM, D), lambda i: (2 * i + 1, 0, 0)),
            pl.BlockSpec((E, D // 128, 128), lambda i: (0, 0, 0)),
            pl.BlockSpec((1, E), lambda i: (0, 0)),
        ],
        out_specs=pl.BlockSpec((E, 2 * _BM), lambda i: (0, i)),
        out_shape=jax.ShapeDtypeStruct((E, B), jnp.float32),
        compiler_params=pltpu.CompilerParams(
            dimension_semantics=("arbitrary",),
        ),
    )(x3, x3, w2, b_row)
    return out_t.T


# speedup vs baseline: 1.1041x; 1.0123x over previous
import jax
import jax.numpy as jnp
from jax import lax
from jax.experimental import pallas as pl
from jax.experimental.pallas import tpu as pltpu

_BM = 1024
_NB = 3


def _gemm_chunk(x, w_ref, bias_t):
    acc = bias_t
    for k in range(w_ref.shape[1]):
        acc = acc + lax.dot_general(
            w_ref[:, k, :],
            x[:, 128 * k : 128 * (k + 1)],
            dimension_numbers=(((1,), (1,)), ((), ())),
            preferred_element_type=jnp.float32,
        )
    return acc


def _make_kernel(n_chunks):
    def _kernel(x_hbm, w_ref, b_ref, o_hbm, xbuf, ostage, xsem, osem):
        bias_t = lax.transpose(b_ref[...], (1, 0))

        def xcopy(i):
            return pltpu.make_async_copy(
                x_hbm.at[pl.ds(i * _BM, _BM), :], xbuf.at[i % _NB], xsem.at[i % _NB]
            )

        def ocopy(i):
            return pltpu.make_async_copy(
                ostage.at[i % 2], o_hbm.at[:, pl.ds(i * _BM, _BM)], osem.at[i % 2]
            )

        for s in range(min(_NB - 1, n_chunks)):
            xcopy(s).start()
        for i in range(n_chunks):
            xcopy(i).wait()
            nxt = i + _NB - 1
            if nxt < n_chunks:
                xcopy(nxt).start()
            acc = _gemm_chunk(xbuf[i % _NB], w_ref, bias_t)
            if i >= 2:
                ocopy(i - 2).wait()
            ostage[i % 2] = acc
            ocopy(i).start()
        for i in range(max(n_chunks - 2, 0), n_chunks):
            ocopy(i).wait()

    return _kernel


def kernel(x, W, b):
    B, D = x.shape
    E = W.shape[0]
    w2 = W.reshape(E, D // 128, 128)
    b_row = b.reshape(1, E)
    out_t = pl.pallas_call(
        _make_kernel(B // _BM),
        in_specs=[
            pl.BlockSpec(memory_space=pltpu.MemorySpace.HBM),
            pl.BlockSpec((E, D // 128, 128), lambda: (0, 0, 0)),
            pl.BlockSpec((1, E), lambda: (0, 0)),
        ],
        out_specs=pl.BlockSpec(memory_space=pltpu.MemorySpace.HBM),
        out_shape=jax.ShapeDtypeStruct((E, B), jnp.float32),
        scratch_shapes=[
            pltpu.VMEM((_NB, _BM, D), jnp.float32),
            pltpu.VMEM((2, E, _BM), jnp.float32),
            pltpu.SemaphoreType.DMA((_NB,)),
            pltpu.SemaphoreType.DMA((2,)),
        ],
    )(x, w2, b_row)
    return out_t.T


# speedup vs baseline: 1.1402x; 1.0327x over previous
import jax
import jax.numpy as jnp
from jax import lax
from jax.experimental import pallas as pl
from jax.experimental.pallas import tpu as pltpu

_BM = 1024


def _gemm_bias_kernel(x_ref, w_ref, b_ref, o_ref):
    x = x_ref[...]
    acc = lax.transpose(b_ref[...], (1, 0))
    for k in range(w_ref.shape[1]):
        acc = acc + lax.dot_general(
            w_ref[:, k, :],
            x[:, 128 * k : 128 * (k + 1)],
            dimension_numbers=(((1,), (1,)), ((), ())),
            preferred_element_type=jnp.float32,
        )
    i = pl.program_id(0)
    o_ref[:, pl.ds(pl.multiple_of(i * _BM, _BM), _BM)] = acc


def kernel(x, W, b):
    B, D = x.shape
    E = W.shape[0]
    w2 = W.reshape(E, D // 128, 128)
    b_row = b.reshape(1, E)
    out_t = pl.pallas_call(
        _gemm_bias_kernel,
        grid=(B // _BM,),
        in_specs=[
            pl.BlockSpec((_BM, D), lambda i: (i, 0)),
            pl.BlockSpec((E, D // 128, 128), lambda i: (0, 0, 0)),
            pl.BlockSpec((1, E), lambda i: (0, 0)),
        ],
        out_specs=pl.BlockSpec((E, B), lambda i: (0, 0)),
        out_shape=jax.ShapeDtypeStruct((E, B), jnp.float32),
        compiler_params=pltpu.CompilerParams(
            dimension_semantics=("arbitrary",),
        ),
    )(x, w2, b_row)
    return out_t.T
